# baseline (speedup 1.0000x reference)
import jax
import jax.numpy as jnp
from jax.experimental import pallas as pl
from jax.experimental.pallas import tpu as pltpu

_BM = 400
_BM2 = 1000


def _hop1_kern(adj_ref, seq_ref, w_ref, h1_ref, a16_ref, f_ref):
    t = pl.program_id(0)

    @pl.when(t == 0)
    def _():
        f_ref[...] = jax.lax.dot_general(
            seq_ref[...], w_ref[...],
            (((1,), (1,)), ((), ())),
            preferred_element_type=jnp.float32,
        )

    @pl.when(t >= 1)
    def _():
        a = adj_ref[...]
        a16_ref[...] = a.astype(jnp.bfloat16)
        h1_ref[...] = jnp.dot(
            a, f_ref[...], preferred_element_type=jnp.float32
        ).astype(jnp.bfloat16)


def _hop2_kern(a16_ref, h1_ref, bias_ref, alpha_ref, o_ref):
    v = jnp.dot(a16_ref[...], h1_ref[...],
                preferred_element_type=jnp.float32)
    v = v + bias_ref[...]
    o_ref[...] = jnp.where(v >= 0, v, alpha_ref[0, 0] * v)


def kernel(seq, adj, W_fc, bias, prelu_a):
    n, in_ft = seq.shape
    out_ft = W_fc.shape[0]
    nb = n // _BM

    def stripe_idx(t):
        return (jnp.where(t == 0, 0, t - 1), 0)

    h1, adj16 = pl.pallas_call(
        _hop1_kern,
        grid=(1 + nb,),
        in_specs=[
            pl.BlockSpec((_BM, n), stripe_idx),
            pl.BlockSpec((n, in_ft), lambda t: (0, 0)),
            pl.BlockSpec((out_ft, in_ft), lambda t: (0, 0)),
        ],
        out_specs=[
            pl.BlockSpec((_BM, out_ft), stripe_idx),
            pl.BlockSpec((_BM, n), stripe_idx),
        ],
        out_shape=[
            jax.ShapeDtypeStruct((n, out_ft), jnp.bfloat16),
            jax.ShapeDtypeStruct((n, n), jnp.bfloat16),
        ],
        scratch_shapes=[pltpu.VMEM((n, out_ft), jnp.float32)],
        compiler_params=pltpu.CompilerParams(
            dimension_semantics=("arbitrary",),
        ),
    )(adj, seq, W_fc)

    return pl.pallas_call(
        _hop2_kern,
        grid=(n // _BM2,),
        in_specs=[
            pl.BlockSpec((_BM2, n), lambda i: (i, 0)),
            pl.BlockSpec((n, out_ft), lambda i: (0, 0)),
            pl.BlockSpec((1, out_ft), lambda i: (0, 0)),
            pl.BlockSpec((1, 1), lambda i: (0, 0)),
        ],
        out_specs=pl.BlockSpec((_BM2, out_ft), lambda i: (i, 0)),
        out_shape=jax.ShapeDtypeStruct((n, out_ft), jnp.float32),
        compiler_params=pltpu.CompilerParams(
            dimension_semantics=("arbitrary",),
        ),
    )(adj16, h1, bias.reshape(1, out_ft), prelu_a.reshape(1, 1))


# speedup vs baseline: 1.0558x; 1.0558x over previous
import jax
import jax.numpy as jnp
from jax.experimental import pallas as pl
from jax.experimental.pallas import tpu as pltpu

_BM = 400


def _gcn_kern(adj_ref, seq_ref, w_ref, bias_ref, alpha_ref, o_ref,
              f_ref, h1_ref):
    t = pl.program_id(0)
    nb = (pl.num_programs(0) - 1) // 2

    @pl.when(t == 0)
    def _():
        f_ref[...] = jax.lax.dot_general(
            seq_ref[...], w_ref[...],
            (((1,), (1,)), ((), ())),
            preferred_element_type=jnp.float32,
        )

    @pl.when(jnp.logical_and(t >= 1, t <= nb))
    def _():
        h1_ref[pl.ds((t - 1) * _BM, _BM), :] = jnp.dot(
            adj_ref[...], f_ref[...], preferred_element_type=jnp.float32)

    @pl.when(t > nb)
    def _():
        v = jnp.dot(adj_ref[...], h1_ref[...],
                    preferred_element_type=jnp.float32)
        v = v + bias_ref[...]
        o_ref[...] = jnp.where(v >= 0, v, alpha_ref[0, 0] * v)


def kernel(seq, adj, W_fc, bias, prelu_a):
    n, in_ft = seq.shape
    out_ft = W_fc.shape[0]
    nb = n // _BM

    def adj_idx(t):
        return (jnp.where(t == 0, 0, jnp.where(t <= nb, t - 1, t - 1 - nb)), 0)

    def out_idx(t):
        return (jnp.where(t <= nb, 0, t - 1 - nb), 0)

    return pl.pallas_call(
        _gcn_kern,
        grid=(1 + 2 * nb,),
        in_specs=[
            pl.BlockSpec((_BM, n), adj_idx),
            pl.BlockSpec((n, in_ft), lambda t: (0, 0)),
            pl.BlockSpec((out_ft, in_ft), lambda t: (0, 0)),
            pl.BlockSpec((1, out_ft), lambda t: (0, 0)),
            pl.BlockSpec((1, 1), lambda t: (0, 0)),
        ],
        out_specs=pl.BlockSpec((_BM, out_ft), out_idx),
        out_shape=jax.ShapeDtypeStruct((n, out_ft), jnp.float32),
        scratch_shapes=[
            pltpu.VMEM((n, out_ft), jnp.float32),
            pltpu.VMEM((n, out_ft), jnp.float32),
        ],
        compiler_params=pltpu.CompilerParams(
            dimension_semantics=("arbitrary",),
        ),
    )(adj, seq, W_fc, bias.reshape(1, out_ft), prelu_a.reshape(1, 1))


# speedup vs baseline: 1.0581x; 1.0021x over previous
import jax
import jax.numpy as jnp
from jax.experimental import pallas as pl
from jax.experimental.pallas import tpu as pltpu

_BM = 400


def _gcn_kern(adj_ref, seq_ref, w_ref, bias_ref, alpha_ref, o_ref,
              f_ref, h1_ref):
    t = pl.program_id(0)
    nb = (pl.num_programs(0) - 1) // 2

    @pl.when(t == 0)
    def _():
        f_ref[...] = jax.lax.dot_general(
            seq_ref[...], w_ref[...],
            (((1,), (1,)), ((), ())),
            preferred_element_type=jnp.float32,
        )

    @pl.when(jnp.logical_and(t >= 1, t <= nb))
    def _():
        h1_ref[pl.ds((t - 1) * _BM, _BM), :] = jnp.dot(
            adj_ref[...], f_ref[...], preferred_element_type=jnp.float32)

    @pl.when(t > nb)
    def _():
        v = jnp.dot(adj_ref[...], h1_ref[...],
                    preferred_element_type=jnp.float32)
        v = v + bias_ref[...]
        o_ref[...] = jnp.where(v >= 0, v, alpha_ref[0, 0] * v)


def kernel(seq, adj, W_fc, bias, prelu_a):
    n, in_ft = seq.shape
    out_ft = W_fc.shape[0]
    nb = n // _BM

    def adj_idx(t):
        return (jnp.where(t == 0, 0, jnp.where(t <= nb, t - 1, 2 * nb - t)), 0)

    def out_idx(t):
        return (jnp.where(t <= nb, nb - 1, 2 * nb - t), 0)

    return pl.pallas_call(
        _gcn_kern,
        grid=(1 + 2 * nb,),
        in_specs=[
            pl.BlockSpec((_BM, n), adj_idx),
            pl.BlockSpec((n, in_ft), lambda t: (0, 0)),
            pl.BlockSpec((out_ft, in_ft), lambda t: (0, 0)),
            pl.BlockSpec((1, out_ft), lambda t: (0, 0)),
            pl.BlockSpec((1, 1), lambda t: (0, 0)),
        ],
        out_specs=pl.BlockSpec((_BM, out_ft), out_idx),
        out_shape=jax.ShapeDtypeStruct((n, out_ft), jnp.float32),
        scratch_shapes=[
            pltpu.VMEM((n, out_ft), jnp.float32),
            pltpu.VMEM((n, out_ft), jnp.float32),
        ],
        compiler_params=pltpu.CompilerParams(
            dimension_semantics=("arbitrary",),
        ),
    )(adj, seq, W_fc, bias.reshape(1, out_ft), prelu_a.reshape(1, 1))


# speedup vs baseline: 1.0662x; 1.0077x over previous
import jax
import jax.numpy as jnp
from jax.experimental import pallas as pl
from jax.experimental.pallas import tpu as pltpu

_BM = 400


def _gcn_kern(adj_ref, seq_ref, w_ref, bias_ref, alpha_ref, o_ref,
              f_ref, h1_ref):
    t = pl.program_id(0)
    nb = pl.num_programs(0) // 2

    @pl.when(t == 0)
    def _():
        f_ref[...] = jax.lax.dot_general(
            seq_ref[...], w_ref[...],
            (((1,), (1,)), ((), ())),
            preferred_element_type=jnp.float32,
        )

    @pl.when(t < nb)
    def _():
        h1_ref[pl.ds(t * _BM, _BM), :] = jnp.dot(
            adj_ref[...], f_ref[...], preferred_element_type=jnp.float32)

    @pl.when(t >= nb)
    def _():
        v = jnp.dot(adj_ref[...], h1_ref[...],
                    preferred_element_type=jnp.float32)
        v = v + bias_ref[...]
        o_ref[...] = jnp.where(v >= 0, v, alpha_ref[0, 0] * v)


def kernel(seq, adj, W_fc, bias, prelu_a):
    n, in_ft = seq.shape
    out_ft = W_fc.shape[0]
    nb = n // _BM

    def adj_idx(t):
        return (jnp.where(t < nb, t, 2 * nb - 1 - t), 0)

    def out_idx(t):
        return (jnp.where(t < nb, nb - 1, 2 * nb - 1 - t), 0)

    return pl.pallas_call(
        _gcn_kern,
        grid=(2 * nb,),
        in_specs=[
            pl.BlockSpec((_BM, n), adj_idx),
            pl.BlockSpec((n, in_ft), lambda t: (0, 0)),
            pl.BlockSpec((out_ft, in_ft), lambda t: (0, 0)),
            pl.BlockSpec((1, out_ft), lambda t: (0, 0)),
            pl.BlockSpec((1, 1), lambda t: (0, 0)),
        ],
        out_specs=pl.BlockSpec((_BM, out_ft), out_idx),
        out_shape=jax.ShapeDtypeStruct((n, out_ft), jnp.float32),
        scratch_shapes=[
            pltpu.VMEM((n, out_ft), jnp.float32),
            pltpu.VMEM((n, out_ft), jnp.float32),
        ],
        compiler_params=pltpu.CompilerParams(
            dimension_semantics=("arbitrary",),
        ),
    )(adj, seq, W_fc, bias.reshape(1, out_ft), prelu_a.reshape(1, 1))


# speedup vs baseline: 1.0757x; 1.0089x over previous
import jax
import jax.numpy as jnp
from jax.experimental import pallas as pl
from jax.experimental.pallas import tpu as pltpu

_BM = 400


def _gcn_kern(adj_ref, seq_ref, w_ref, bias_ref, alpha_ref, o_ref,
              f_ref, h1_ref):
    t = pl.program_id(0)
    nb = (pl.num_programs(0) + 1) // 2

    @pl.when(t == 0)
    def _():
        f_ref[...] = jax.lax.dot_general(
            seq_ref[...], w_ref[...],
            (((1,), (1,)), ((), ())),
            preferred_element_type=jnp.float32,
        )

    @pl.when(t < nb)
    def _():
        h1_ref[pl.ds(t * _BM, _BM), :] = jnp.dot(
            adj_ref[...], f_ref[...], preferred_element_type=jnp.float32)

    @pl.when(t >= nb - 1)
    def _():
        v = jnp.dot(adj_ref[...], h1_ref[...],
                    preferred_element_type=jnp.float32)
        v = v + bias_ref[...]
        o_ref[...] = jnp.where(v >= 0, v, alpha_ref[0, 0] * v)


def kernel(seq, adj, W_fc, bias, prelu_a):
    n, in_ft = seq.shape
    out_ft = W_fc.shape[0]
    nb = n // _BM

    def adj_idx(t):
        return (jnp.where(t < nb, t, 2 * nb - 2 - t), 0)

    def out_idx(t):
        return (jnp.where(t < nb, nb - 1, 2 * nb - 2 - t), 0)

    return pl.pallas_call(
        _gcn_kern,
        grid=(2 * nb - 1,),
        in_specs=[
            pl.BlockSpec((_BM, n), adj_idx),
            pl.BlockSpec((n, in_ft), lambda t: (0, 0)),
            pl.BlockSpec((out_ft, in_ft), lambda t: (0, 0)),
            pl.BlockSpec((1, out_ft), lambda t: (0, 0)),
            pl.BlockSpec((1, 1), lambda t: (0, 0)),
        ],
        out_specs=pl.BlockSpec((_BM, out_ft), out_idx),
        out_shape=jax.ShapeDtypeStruct((n, out_ft), jnp.float32),
        scratch_shapes=[
            pltpu.VMEM((n, out_ft), jnp.float32),
            pltpu.VMEM((n, out_ft), jnp.float32),
        ],
        compiler_params=pltpu.CompilerParams(
            dimension_semantics=("arbitrary",),
        ),
    )(adj, seq, W_fc, bias.reshape(1, out_ft), prelu_a.reshape(1, 1))


# speedup vs baseline: 1.0758x; 1.0001x over previous
import jax
import jax.numpy as jnp
from jax.experimental import pallas as pl
from jax.experimental.pallas import tpu as pltpu

_BM = 400


def _gcn_kern(adj_ref, seq_ref, w_ref, bias_ref, alpha_ref, o_ref,
              f_ref, h1_ref):
    t = pl.program_id(0)
    nb = (pl.num_programs(0) + 1) // 2

    @pl.when(t == 0)
    def _():
        f_ref[...] = jax.lax.dot_general(
            seq_ref[...], w_ref[...],
            (((1,), (1,)), ((), ())),
            preferred_element_type=jnp.float32,
        )

    @pl.when(t < nb)
    def _():
        row = jnp.where(t < nb - 2, t, 2 * nb - 3 - t)
        h1_ref[pl.ds(row * _BM, _BM), :] = jnp.dot(
            adj_ref[...], f_ref[...], preferred_element_type=jnp.float32)

    @pl.when(t >= nb - 1)
    def _():
        v = jnp.dot(adj_ref[...], h1_ref[...],
                    preferred_element_type=jnp.float32)
        v = v + bias_ref[...]
        o_ref[...] = jnp.where(v >= 0, v, alpha_ref[0, 0] * v)


def kernel(seq, adj, W_fc, bias, prelu_a):
    n, in_ft = seq.shape
    out_ft = W_fc.shape[0]
    nb = n // _BM

    def adj_idx(t):
        return (
            jnp.where(
                t < nb - 2, t,
                jnp.where(t < nb, 2 * nb - 3 - t,
                          jnp.where(t == nb, nb - 1, 2 * nb - 2 - t))),
            0,
        )

    def out_idx(t):
        return (
            jnp.where(t < nb, nb - 2,
                      jnp.where(t == nb, nb - 1, 2 * nb - 2 - t)),
            0,
        )

    return pl.pallas_call(
        _gcn_kern,
        grid=(2 * nb - 1,),
        in_specs=[
            pl.BlockSpec((_BM, n), adj_idx),
            pl.BlockSpec((n, in_ft), lambda t: (0, 0)),
            pl.BlockSpec((out_ft, in_ft), lambda t: (0, 0)),
            pl.BlockSpec((1, out_ft), lambda t: (0, 0)),
            pl.BlockSpec((1, 1), lambda t: (0, 0)),
        ],
        out_specs=pl.BlockSpec((_BM, out_ft), out_idx),
        out_shape=jax.ShapeDtypeStruct((n, out_ft), jnp.float32),
        scratch_shapes=[
            pltpu.VMEM((n, out_ft), jnp.float32),
            pltpu.VMEM((n, out_ft), jnp.float32),
        ],
        compiler_params=pltpu.CompilerParams(
            dimension_semantics=("arbitrary",),
        ),
    )(adj, seq, W_fc, bias.reshape(1, out_ft), prelu_a.reshape(1, 1))
